# baseline (speedup 1.0000x reference)
import functools

import jax
import jax.numpy as jnp
from jax import lax
from jax.experimental import pallas as pl
from jax.experimental.pallas import tpu as pltpu
from jax.experimental.pallas import tpu_sc as plsc

N = 10000
E = 320000
D = 128
EPS = 1e-5

NC = 2
NS = 16
L = 16

NP = 10112
ND = 10240
K = 80
C = 125
EPW = C * K
RP = NP // NS
RD = ND // NS

_mesh = plsc.VectorSubcoreMesh(core_axis_name="c", subcore_axis_name="s")


def _drain(dummy_src, dst_ref, sem):
    pltpu.make_async_copy(dummy_src, dst_ref, sem).wait()


@functools.partial(
    pl.kernel,
    out_type=jax.ShapeDtypeStruct((NC, ND), jnp.float32),
    scratch_types=[
        pltpu.VMEM_SHARED((ND,), jnp.float32),
        pltpu.VMEM((8, K), jnp.int32),
        pltpu.VMEM((RD,), jnp.float32),
        pltpu.VMEM((K,), jnp.float32),
        pltpu.SemaphoreType.DMA((8,)),
        pltpu.SemaphoreType.DMA((8,)),
    ],
    mesh=_mesh,
)
def _deg_kernel(dst_hbm, out_hbm, deg_sh, didx, zero_v, ones_v, isem, ssem):
    c = lax.axis_index("c")
    s = lax.axis_index("s")
    w = s * NC + c

    zeros = jnp.zeros((L,), jnp.float32)

    def zbody(i, carry):
        zero_v[pl.ds(pl.multiple_of(i * L, L), L)] = zeros
        return carry

    lax.fori_loop(0, RD // L, zbody, 0)
    for k in range(K // L):
        ones_v[pl.ds(k * L, L)] = jnp.ones((L,), jnp.float32)

    pltpu.sync_copy(zero_v, deg_sh.at[pl.ds(s * RD, RD)])
    plsc.subcore_barrier()

    base = w * EPW

    def emit(j, b, wait_pre, do_pre):
        if do_pre:
            pb = (b + 4) % 8
            if wait_pre:
                _drain(dst_hbm.at[pl.ds(0, K)], didx.at[pb], ssem.at[pb])
            pltpu.async_copy(
                dst_hbm.at[pl.ds(base + (j + 4) * K, K)], didx.at[pb],
                isem.at[pb])
        _drain(dst_hbm.at[pl.ds(0, K)], didx.at[b], isem.at[b])
        pltpu.async_copy(ones_v, deg_sh.at[didx.at[b]], ssem.at[b], add=True)

    for t in range(4):
        pltpu.async_copy(dst_hbm.at[pl.ds(base + t * K, K)], didx.at[t],
                         isem.at[t])
    for j in range(8):
        emit(j, j, wait_pre=j >= 4, do_pre=True)

    def body(j8, carry):
        for b in range(8):
            emit(j8 * 8 + b, b, wait_pre=True, do_pre=True)
        return carry

    lax.fori_loop(1, C // 8, body, 0)
    for j in range((C // 8) * 8, C):
        emit(j, j % 8, wait_pre=True, do_pre=j + 4 < C)
    for b in range(8):
        _drain(dst_hbm.at[pl.ds(0, K)], didx.at[b], ssem.at[b])

    plsc.subcore_barrier()
    pltpu.sync_copy(deg_sh.at[pl.ds(s * RD, RD)], out_hbm.at[c, pl.ds(s * RD, RD)])


@functools.partial(
    pl.kernel,
    out_type=jax.ShapeDtypeStruct((NC, NP, D), jnp.float32),
    scratch_types=[
        pltpu.VMEM_SHARED((NP, D), jnp.float32),
        pltpu.VMEM((8, K), jnp.int32),
        pltpu.VMEM((8, K), jnp.int32),
        pltpu.VMEM((4, K, D), jnp.float32),
        pltpu.SemaphoreType.DMA((8,)),
        pltpu.SemaphoreType.DMA((8,)),
        pltpu.SemaphoreType.DMA((4,)),
        pltpu.SemaphoreType.DMA((8,)),
    ],
    mesh=_mesh,
)
def _agg_kernel(g_hbm, src_hbm, dst_hbm, out_hbm, acc_sh, sidx, didx, rows,
                issem, idsem, gsem, ssem):
    c = lax.axis_index("c")
    s = lax.axis_index("s")
    w = s * NC + c
    pltpu.sync_copy(g_hbm.at[pl.ds(s * RP, RP)], acc_sh.at[pl.ds(s * RP, RP)])
    plsc.subcore_barrier()

    base = w * EPW

    def idx_dma(t, slot):
        off = base + t * K
        pltpu.async_copy(src_hbm.at[pl.ds(off, K)], sidx.at[slot],
                         issem.at[slot])
        pltpu.async_copy(dst_hbm.at[pl.ds(off, K)], didx.at[slot],
                         idsem.at[slot])

    def emit(j, b, wait_sct, do_pre, do_gath):
        if do_pre:
            idx_dma(j + 4, (b + 4) % 8)
        if do_gath:
            nb = (b + 2) % 8
            if wait_sct:
                _drain(g_hbm.at[pl.ds(0, K)], rows.at[(b + 2) % 4],
                       ssem.at[(b + 6) % 8])
            _drain(src_hbm.at[pl.ds(0, K)], sidx.at[nb], issem.at[nb])
            pltpu.async_copy(g_hbm.at[sidx.at[nb]], rows.at[nb % 4],
                             gsem.at[nb % 4])
        _drain(g_hbm.at[pl.ds(0, K)], rows.at[b % 4], gsem.at[b % 4])
        _drain(dst_hbm.at[pl.ds(0, K)], didx.at[b], idsem.at[b])
        pltpu.async_copy(rows.at[b % 4], acc_sh.at[didx.at[b]], ssem.at[b],
                         add=True)

    for t in range(4):
        idx_dma(t, t)
    for t in range(2):
        _drain(src_hbm.at[pl.ds(0, K)], sidx.at[t], issem.at[t])
        pltpu.async_copy(g_hbm.at[sidx.at[t]], rows.at[t], gsem.at[t])

    for j in range(8):
        emit(j, j, wait_sct=j >= 2, do_pre=True, do_gath=True)

    def body(j8, carry):
        for b in range(8):
            emit(j8 * 8 + b, b, wait_sct=True, do_pre=True, do_gath=True)
        return carry

    lax.fori_loop(1, C // 8, body, 0)
    for j in range((C // 8) * 8, C):
        emit(j, j % 8, wait_sct=True, do_pre=j + 4 < C, do_gath=j + 2 < C)
    for q in range(C - 4, C):
        _drain(g_hbm.at[pl.ds(0, K)], rows.at[q % 4], ssem.at[q % 8])

    plsc.subcore_barrier()
    pltpu.sync_copy(acc_sh.at[pl.ds(s * RP, RP)], out_hbm.at[c, pl.ds(s * RP, RP)])


def _mm_body(x_ref, w_ref, h_ref):
    h_ref[...] = jnp.dot(x_ref[...], w_ref[...],
                         preferred_element_type=jnp.float32)


def _scale_body(degp_ref, h_ref, g_ref, dis_ref):
    deg = jnp.sum(degp_ref[...], axis=0)[:NP] + 1.0
    dis = lax.rsqrt(deg)
    dis_ref[...] = dis
    g_ref[:N] = h_ref[...] * dis[:N, None]


def _bn_body(agg_ref, g_ref, dis_ref, b_ref, gamma_ref, beta_ref, out_ref):
    dis = dis_ref[...][:N]
    acc = agg_ref[0, :N, :] + agg_ref[1, :N, :] - g_ref[:N, :]
    pre = acc * dis[:, None] + b_ref[...]
    mean = jnp.mean(pre, axis=0)
    cen = pre - mean
    var = jnp.mean(cen * cen, axis=0)
    y = cen * lax.rsqrt(var + EPS) * gamma_ref[...] + beta_ref[...]
    out_ref[...] = jnp.maximum(y, 0.0)


def kernel(x, edge_index, W, b, gamma, beta):
    src = edge_index[0].astype(jnp.int32)
    dst = edge_index[1].astype(jnp.int32)

    deg_part = _deg_kernel(dst)
    h = pl.pallas_call(
        _mm_body,
        out_shape=jax.ShapeDtypeStruct((N, D), jnp.float32),
    )(x, W)
    g, dis = pl.pallas_call(
        _scale_body,
        out_shape=(
            jax.ShapeDtypeStruct((NP, D), jnp.float32),
            jax.ShapeDtypeStruct((NP,), jnp.float32),
        ),
    )(deg_part, h)
    agg = _agg_kernel(g, src, dst)
    out = pl.pallas_call(
        _bn_body,
        out_shape=jax.ShapeDtypeStruct((N, D), jnp.float32),
    )(agg, g, dis, b, gamma, beta)
    return out


# speedup vs baseline: 1.0087x; 1.0087x over previous
import functools

import jax
import jax.numpy as jnp
from jax import lax
from jax.experimental import pallas as pl
from jax.experimental.pallas import tpu as pltpu
from jax.experimental.pallas import tpu_sc as plsc

N = 10000
E = 320000
D = 128
EPS = 1e-5

NC = 2
NS = 16
L = 16

NP = 10112
ND = 10240
K = 80
C = 125
EPW = C * K
RP = NP // NS
RD = ND // NS

_mesh = plsc.VectorSubcoreMesh(core_axis_name="c", subcore_axis_name="s")


def _drain(dummy_src, dst_ref, sem):
    pltpu.make_async_copy(dummy_src, dst_ref, sem).wait()


@functools.partial(
    pl.kernel,
    out_type=jax.ShapeDtypeStruct((NC, ND), jnp.float32),
    scratch_types=[
        pltpu.VMEM_SHARED((ND,), jnp.float32),
        pltpu.VMEM((8, K), jnp.int32),
        pltpu.VMEM((RD,), jnp.float32),
        pltpu.VMEM((K,), jnp.float32),
        pltpu.SemaphoreType.DMA((8,)),
        pltpu.SemaphoreType.DMA((8,)),
    ],
    mesh=_mesh,
)
def _deg_kernel(dst_hbm, out_hbm, deg_sh, didx, zero_v, ones_v, isem, ssem):
    c = lax.axis_index("c")
    s = lax.axis_index("s")
    w = s * NC + c

    zeros = jnp.zeros((L,), jnp.float32)

    def zbody(i, carry):
        zero_v[pl.ds(pl.multiple_of(i * L, L), L)] = zeros
        return carry

    lax.fori_loop(0, RD // L, zbody, 0)
    for k in range(K // L):
        ones_v[pl.ds(k * L, L)] = jnp.ones((L,), jnp.float32)

    pltpu.sync_copy(zero_v, deg_sh.at[pl.ds(s * RD, RD)])
    plsc.subcore_barrier()

    base = w * EPW

    def emit(j, b, wait_pre, do_pre):
        if do_pre:
            pb = (b + 4) % 8
            if wait_pre:
                _drain(dst_hbm.at[pl.ds(0, K)], didx.at[pb], ssem.at[pb])
            pltpu.async_copy(
                dst_hbm.at[pl.ds(base + (j + 4) * K, K)], didx.at[pb],
                isem.at[pb])
        _drain(dst_hbm.at[pl.ds(0, K)], didx.at[b], isem.at[b])
        pltpu.async_copy(ones_v, deg_sh.at[didx.at[b]], ssem.at[b], add=True)

    for t in range(4):
        pltpu.async_copy(dst_hbm.at[pl.ds(base + t * K, K)], didx.at[t],
                         isem.at[t])
    for j in range(8):
        emit(j, j, wait_pre=j >= 4, do_pre=True)

    def body(j8, carry):
        for b in range(8):
            emit(j8 * 8 + b, b, wait_pre=True, do_pre=True)
        return carry

    lax.fori_loop(1, C // 8, body, 0)
    for j in range((C // 8) * 8, C):
        emit(j, j % 8, wait_pre=True, do_pre=j + 4 < C)
    for b in range(8):
        _drain(dst_hbm.at[pl.ds(0, K)], didx.at[b], ssem.at[b])

    plsc.subcore_barrier()
    pltpu.sync_copy(deg_sh.at[pl.ds(s * RD, RD)], out_hbm.at[c, pl.ds(s * RD, RD)])


@functools.partial(
    pl.kernel,
    out_type=jax.ShapeDtypeStruct((NC, NP, D), jnp.float32),
    scratch_types=[
        pltpu.VMEM_SHARED((NP, D), jnp.float32),
        pltpu.VMEM((8, K), jnp.int32),
        pltpu.VMEM((8, K), jnp.int32),
        pltpu.VMEM((4, K, D), jnp.float32),
        pltpu.SemaphoreType.DMA((8,)),
        pltpu.SemaphoreType.DMA((8,)),
        pltpu.SemaphoreType.DMA((4,)),
        pltpu.SemaphoreType.DMA((8,)),
    ],
    mesh=_mesh,
)
def _agg_kernel(g_hbm, src_hbm, dst_hbm, out_hbm, acc_sh, sidx, didx, rows,
                issem, idsem, gsem, ssem):
    c = lax.axis_index("c")
    s = lax.axis_index("s")
    w = s * NC + c
    pltpu.sync_copy(g_hbm.at[pl.ds(s * RP, RP)], acc_sh.at[pl.ds(s * RP, RP)])
    plsc.subcore_barrier()

    base = w * EPW

    def idx_dma(t, slot):
        off = base + t * K
        pltpu.async_copy(src_hbm.at[pl.ds(off, K)], sidx.at[slot],
                         issem.at[slot])
        pltpu.async_copy(dst_hbm.at[pl.ds(off, K)], didx.at[slot],
                         idsem.at[slot])

    def emit(j, b, wait_sct, do_pre, do_gath):
        if do_pre:
            idx_dma(j + 4, (b + 4) % 8)
        if do_gath:
            nb = (b + 2) % 8
            if wait_sct:
                _drain(g_hbm.at[pl.ds(0, K)], rows.at[(b + 2) % 4],
                       ssem.at[(b + 6) % 8])
            _drain(src_hbm.at[pl.ds(0, K)], sidx.at[nb], issem.at[nb])
            pltpu.async_copy(g_hbm.at[sidx.at[nb]], rows.at[nb % 4],
                             gsem.at[nb % 4])
        _drain(g_hbm.at[pl.ds(0, K)], rows.at[b % 4], gsem.at[b % 4])
        _drain(dst_hbm.at[pl.ds(0, K)], didx.at[b], idsem.at[b])
        pltpu.async_copy(rows.at[b % 4], acc_sh.at[didx.at[b]], ssem.at[b],
                         add=True)

    for t in range(4):
        idx_dma(t, t)
    for t in range(2):
        _drain(src_hbm.at[pl.ds(0, K)], sidx.at[t], issem.at[t])
        pltpu.async_copy(g_hbm.at[sidx.at[t]], rows.at[t], gsem.at[t])

    for j in range(8):
        emit(j, j, wait_sct=j >= 2, do_pre=True, do_gath=True)

    def body(j8, carry):
        for b in range(8):
            emit(j8 * 8 + b, b, wait_sct=True, do_pre=True, do_gath=True)
        return carry

    lax.fori_loop(1, C // 8, body, 0)
    for j in range((C // 8) * 8, C):
        emit(j, j % 8, wait_sct=True, do_pre=j + 4 < C, do_gath=j + 2 < C)
    for q in range(C - 4, C):
        _drain(g_hbm.at[pl.ds(0, K)], rows.at[q % 4], ssem.at[q % 8])

    plsc.subcore_barrier()
    pltpu.sync_copy(acc_sh.at[pl.ds(s * RP, RP)], out_hbm.at[c, pl.ds(s * RP, RP)])


def _mm_body(degp_ref, x_ref, w_ref, g_ref, dis_ref):
    deg = jnp.sum(degp_ref[...], axis=0)[:NP] + 1.0
    dis = lax.rsqrt(deg)
    dis_ref[...] = dis
    xs = x_ref[...] * dis[:N, None]
    g_ref[:N] = jnp.dot(xs, w_ref[...], preferred_element_type=jnp.float32)


def _bn_body(agg_ref, g_ref, dis_ref, b_ref, gamma_ref, beta_ref, out_ref):
    dis = dis_ref[...][:N]
    acc = agg_ref[0, :N, :] + agg_ref[1, :N, :] - g_ref[:N, :]
    pre = acc * dis[:, None] + b_ref[...]
    mean = jnp.mean(pre, axis=0)
    cen = pre - mean
    var = jnp.mean(cen * cen, axis=0)
    y = cen * lax.rsqrt(var + EPS) * gamma_ref[...] + beta_ref[...]
    out_ref[...] = jnp.maximum(y, 0.0)


def kernel(x, edge_index, W, b, gamma, beta):
    src = edge_index[0].astype(jnp.int32)
    dst = edge_index[1].astype(jnp.int32)

    deg_part = _deg_kernel(dst)
    g, dis = pl.pallas_call(
        _mm_body,
        out_shape=(
            jax.ShapeDtypeStruct((NP, D), jnp.float32),
            jax.ShapeDtypeStruct((NP,), jnp.float32),
        ),
    )(deg_part, x, W)
    agg = _agg_kernel(g, src, dst)
    out = pl.pallas_call(
        _bn_body,
        out_shape=jax.ShapeDtypeStruct((N, D), jnp.float32),
    )(agg, g, dis, b, gamma, beta)
    return out
